# baseline (speedup 1.0000x reference)
import functools

import jax
import jax.numpy as jnp
from jax import lax
from jax.experimental import pallas as pl
from jax.experimental.pallas import tpu as pltpu
from jax.experimental.pallas import tpu_sc as plsc

_NC = 2
_NS = 16
_NW = _NC * _NS


def _sc_gather_fn(n_tokens, hidden, chunk):
    n_per_w = n_tokens // _NW
    n_chunks = n_per_w // chunk

    mesh = plsc.VectorSubcoreMesh(core_axis_name="c", subcore_axis_name="s")

    @functools.partial(
        pl.kernel,
        out_type=jax.ShapeDtypeStruct((n_tokens, hidden), jnp.float32),
        mesh=mesh,
        scratch_types=[
            pltpu.VMEM((chunk,), jnp.int32),
            pltpu.VMEM((chunk, hidden), jnp.float32),
            pltpu.SemaphoreType.DMA,
        ],
    )
    def sc_gather(idx_hbm, table_hbm, x_hbm, idx_v, rows_v, sem):
        wid = lax.axis_index("s") * _NC + lax.axis_index("c")
        base = wid * n_per_w
        for c in range(n_chunks):
            off = base + c * chunk
            pltpu.sync_copy(idx_hbm.at[pl.ds(off, chunk)], idx_v)
            pltpu.async_copy(table_hbm.at[idx_v], rows_v, sem).wait()
            pltpu.sync_copy(rows_v, x_hbm.at[pl.ds(off, chunk)])

    return sc_gather


def _tc_matmul_body(x_ref, w_ref, b_ref, o_ref):
    o_ref[...] = (
        jnp.dot(x_ref[...], w_ref[...], preferred_element_type=jnp.float32)
        + b_ref[...]
    )


def _tc_matmul(x, w, b2d, block_m):
    n_tokens, hidden = x.shape
    vocab = w.shape[1]
    grid = (n_tokens // block_m,)
    return pl.pallas_call(
        _tc_matmul_body,
        grid=grid,
        in_specs=[
            pl.BlockSpec((block_m, hidden), lambda i: (i, 0)),
            pl.BlockSpec((hidden, vocab), lambda i: (0, 0)),
            pl.BlockSpec((1, vocab), lambda i: (0, 0)),
        ],
        out_specs=pl.BlockSpec((block_m, vocab), lambda i: (i, 0)),
        out_shape=jax.ShapeDtypeStruct((n_tokens, vocab), jnp.float32),
    )(x, w, b2d)


@jax.jit
def kernel(input_ids, embedding, W, b):
    bsz, seqlen = input_ids.shape
    vocab, hidden = embedding.shape
    n_tokens = bsz * seqlen

    ids = input_ids.reshape(-1).astype(jnp.int32)
    x = _sc_gather_fn(n_tokens, hidden, chunk=80)(ids, embedding)
    logits = _tc_matmul(x, W, b.reshape(1, -1), block_m=512)
    return logits.reshape(bsz, seqlen, vocab)
